# baseline (speedup 1.0000x reference)
import jax
import jax.numpy as jnp
from jax import lax
from jax.experimental import pallas as pl
from jax.experimental.pallas import tpu as pltpu

MAX_OBJ = 100
IOU_THRES = 0.45
SCORE_THRES = 0.25
NC = 80
MAX_WH = 640.0
N_BOXES = 20000
NPAD = 20480
NROW = NPAD // 128
NEG = -1.0e30
BIGI = 1 << 30


def _prep_kernel(x_ref, f_ref):
    conf = x_ref[:, 4:5]
    cls = x_ref[:, 5:85]
    scores = conf * cls
    cs = jnp.max(scores, axis=1, keepdims=True)
    csub = lax.broadcasted_iota(jnp.int32, (128, NC), 1).astype(
        jnp.float32)
    ci = (NC - 1) - jnp.max(
        jnp.where(scores == cs, (NC - 1) - csub, -1.0), axis=1,
        keepdims=True)
    bx1 = x_ref[:, 0:1] - x_ref[:, 2:3] * 0.5
    by1 = x_ref[:, 1:2] - x_ref[:, 3:4] * 0.5
    bx2 = x_ref[:, 0:1] + x_ref[:, 2:3] * 0.5
    by2 = x_ref[:, 1:2] + x_ref[:, 3:4] * 0.5
    off = ci * MAX_WH
    ox1 = bx1 + off
    oy1 = by1 + off
    ox2 = bx2 + off
    oy2 = by2 + off
    avail = jnp.where(cs > SCORE_THRES, cs, NEG)
    area = (ox2 - ox1) * (oy2 - oy1)
    zero = jnp.zeros((128, 1), jnp.float32)
    f_ref[...] = jnp.concatenate(
        [avail, ox1, oy1, ox2, oy2, bx1, by1, bx2, by2, ci, area,
         zero, zero, zero, zero, zero], axis=1)


def _nms_kernel(f_ref, dets_ref, av_ref):
    av_ref[...] = f_ref[0]

    ridx = lax.broadcasted_iota(jnp.int32, (NROW, 128), 0)
    lidx = lax.broadcasted_iota(jnp.int32, (NROW, 128), 1)
    idxv = ridx * 128 + lidx
    lane = lax.broadcasted_iota(jnp.int32, (1, 128), 1)

    def body(it, _):
        av = av_ref[...]
        m = jnp.max(av)
        ok = m > 0.0
        wi = jnp.min(jnp.where(av == m, idxv, BIGI))
        sel = idxv == wi

        def pick(r):
            return jnp.max(jnp.where(sel, f_ref[r], NEG))

        rb1, rb2, rb3, rb4 = pick(5), pick(6), pick(7), pick(8)
        wcls = pick(9)
        woff = wcls * MAX_WH
        wx1 = rb1 + woff
        wy1 = rb2 + woff
        wx2 = rb3 + woff
        wy2 = rb4 + woff
        warea = (wx2 - wx1) * (wy2 - wy1)
        ix1 = jnp.maximum(wx1, f_ref[1])
        iy1 = jnp.maximum(wy1, f_ref[2])
        ix2 = jnp.minimum(wx2, f_ref[3])
        iy2 = jnp.minimum(wy2, f_ref[4])
        inter = jnp.maximum(ix2 - ix1, 0.0) * jnp.maximum(iy2 - iy1, 0.0)
        iou = inter / (warea + f_ref[10] - inter + 1e-9)
        supp = jnp.logical_or(jnp.logical_and(ok, iou > IOU_THRES), sel)
        av_ref[...] = jnp.where(supp, NEG, av)

        row = jnp.where(lane == 0, jnp.where(ok, rb1, 0.0), 0.0)
        row = jnp.where(lane == 1, jnp.where(ok, rb2, 0.0), row)
        row = jnp.where(lane == 2, jnp.where(ok, rb3, 0.0), row)
        row = jnp.where(lane == 3, jnp.where(ok, rb4, 0.0), row)
        row = jnp.where(lane == 4, jnp.where(ok, m, 0.0), row)
        row = jnp.where(lane == 5, jnp.where(ok, wcls, -1.0), row)
        dets_ref[pl.ds(it, 1), :] = row
        return 0

    lax.fori_loop(0, MAX_OBJ, body, 0)


def _prep(xp, interpret=False):
    return pl.pallas_call(
        _prep_kernel,
        grid=(NROW,),
        in_specs=[pl.BlockSpec((128, 85), lambda i: (i, 0))],
        out_specs=pl.BlockSpec((128, 16), lambda i: (i, 0)),
        out_shape=jax.ShapeDtypeStruct((NPAD, 16), jnp.float32),
        interpret=interpret,
    )(xp)


def _run_nms(ft, interpret=False):
    return pl.pallas_call(
        _nms_kernel,
        out_shape=jax.ShapeDtypeStruct((MAX_OBJ, 128), jnp.float32),
        scratch_shapes=[pltpu.VMEM((NROW, 128), jnp.float32)],
        interpret=interpret,
    )(ft)


def kernel(x):
    xp = jnp.pad(x[0], ((0, NPAD - N_BOXES), (0, 0)))
    fbm = _prep(xp)
    ft = fbm.T.reshape(16, NROW, 128)[:12]
    dets = _run_nms(ft)
    return dets[None, :, :6]


# speedup vs baseline: 1.0022x; 1.0022x over previous
import jax
import jax.numpy as jnp
from jax import lax
from jax.experimental import pallas as pl
from jax.experimental.pallas import tpu as pltpu

MAX_OBJ = 100
IOU_THRES = 0.45
SCORE_THRES = 0.25
NC = 80
MAX_WH = 640.0
N_BOXES = 20000
NPAD = 20480
NROW = NPAD // 128
NEG = -1.0e30
BIGI = 1 << 30


def _prep_kernel(x_ref, f_ref):
    conf = x_ref[:, 4:5]
    cls = x_ref[:, 5:85]
    scores = conf * cls
    cs = jnp.max(scores, axis=1, keepdims=True)
    csub = lax.broadcasted_iota(jnp.int32, (128, NC), 1).astype(
        jnp.float32)
    ci = (NC - 1) - jnp.max(
        jnp.where(scores == cs, (NC - 1) - csub, -1.0), axis=1,
        keepdims=True)
    bx1 = x_ref[:, 0:1] - x_ref[:, 2:3] * 0.5
    by1 = x_ref[:, 1:2] - x_ref[:, 3:4] * 0.5
    bx2 = x_ref[:, 0:1] + x_ref[:, 2:3] * 0.5
    by2 = x_ref[:, 1:2] + x_ref[:, 3:4] * 0.5
    off = ci * MAX_WH
    ox1 = bx1 + off
    oy1 = by1 + off
    ox2 = bx2 + off
    oy2 = by2 + off
    avail = jnp.where(cs > SCORE_THRES, cs, NEG)
    area = (ox2 - ox1) * (oy2 - oy1)
    zero = jnp.zeros((128, 1), jnp.float32)
    fb = jnp.concatenate(
        [avail, ox1, oy1, ox2, oy2, bx1, by1, bx2, by2, ci, area,
         zero, zero, zero, zero, zero], axis=1)
    f_ref[...] = fb.T


def _nms_kernel(f_ref, dets_ref, av_ref):
    av_ref[...] = f_ref[0]

    ridx = lax.broadcasted_iota(jnp.int32, (NROW, 128), 0)
    lidx = lax.broadcasted_iota(jnp.int32, (NROW, 128), 1)
    idxv = ridx * 128 + lidx
    lane = lax.broadcasted_iota(jnp.int32, (1, 128), 1)

    def body(it, _):
        av = av_ref[...]
        m = jnp.max(av)
        ok = m > 0.0
        wi = jnp.min(jnp.where(av == m, idxv, BIGI))
        sel = idxv == wi

        def pick(r):
            return jnp.max(jnp.where(sel, f_ref[r], NEG))

        rb1, rb2, rb3, rb4 = pick(5), pick(6), pick(7), pick(8)
        wcls = pick(9)
        woff = wcls * MAX_WH
        wx1 = rb1 + woff
        wy1 = rb2 + woff
        wx2 = rb3 + woff
        wy2 = rb4 + woff
        warea = (wx2 - wx1) * (wy2 - wy1)
        ix1 = jnp.maximum(wx1, f_ref[1])
        iy1 = jnp.maximum(wy1, f_ref[2])
        ix2 = jnp.minimum(wx2, f_ref[3])
        iy2 = jnp.minimum(wy2, f_ref[4])
        inter = jnp.maximum(ix2 - ix1, 0.0) * jnp.maximum(iy2 - iy1, 0.0)
        iou = inter / (warea + f_ref[10] - inter + 1e-9)
        supp = jnp.logical_or(jnp.logical_and(ok, iou > IOU_THRES), sel)
        av_ref[...] = jnp.where(supp, NEG, av)

        row = jnp.where(lane == 0, jnp.where(ok, rb1, 0.0), 0.0)
        row = jnp.where(lane == 1, jnp.where(ok, rb2, 0.0), row)
        row = jnp.where(lane == 2, jnp.where(ok, rb3, 0.0), row)
        row = jnp.where(lane == 3, jnp.where(ok, rb4, 0.0), row)
        row = jnp.where(lane == 4, jnp.where(ok, m, 0.0), row)
        row = jnp.where(lane == 5, jnp.where(ok, wcls, -1.0), row)
        dets_ref[pl.ds(it, 1), :] = row
        return 0

    lax.fori_loop(0, MAX_OBJ, body, 0)


def _prep(xp, interpret=False):
    return pl.pallas_call(
        _prep_kernel,
        grid=(NROW,),
        in_specs=[pl.BlockSpec((128, 85), lambda i: (i, 0))],
        out_specs=pl.BlockSpec((16, 128), lambda i: (0, i)),
        out_shape=jax.ShapeDtypeStruct((16, NPAD), jnp.float32),
        interpret=interpret,
    )(xp)


def _run_nms(ft, interpret=False):
    return pl.pallas_call(
        _nms_kernel,
        out_shape=jax.ShapeDtypeStruct((MAX_OBJ, 128), jnp.float32),
        scratch_shapes=[pltpu.VMEM((NROW, 128), jnp.float32)],
        interpret=interpret,
    )(ft)


def kernel(x):
    xp = jnp.pad(x[0], ((0, NPAD - N_BOXES), (0, 0)))
    ft = _prep(xp).reshape(16, NROW, 128)[:12]
    dets = _run_nms(ft)
    return dets[None, :, :6]


# speedup vs baseline: 1.6666x; 1.6629x over previous
import jax
import jax.numpy as jnp
from jax import lax
from jax.experimental import pallas as pl
from jax.experimental.pallas import tpu as pltpu

MAX_OBJ = 100
IOU_THRES = 0.45
SCORE_THRES = 0.25
NC = 80
MAX_WH = 640.0
N_BOXES = 20000
NPAD = 20480
NROW = NPAD // 128
NEG = -1.0e30
BIGI = 1 << 30


BP = 512


def _prep_kernel(x_ref, f_ref):
    xt = x_ref[...].T
    conf = xt[4:5, :]
    cls = xt[5:85, :]
    scores = conf * cls
    cs = jnp.max(scores, axis=0, keepdims=True)
    csub = lax.broadcasted_iota(jnp.int32, (NC, BP), 0).astype(
        jnp.float32)
    ci = (NC - 1) - jnp.max(
        jnp.where(scores == cs, (NC - 1) - csub, -1.0), axis=0,
        keepdims=True)
    bx1 = xt[0:1, :] - xt[2:3, :] * 0.5
    by1 = xt[1:2, :] - xt[3:4, :] * 0.5
    bx2 = xt[0:1, :] + xt[2:3, :] * 0.5
    by2 = xt[1:2, :] + xt[3:4, :] * 0.5
    off = ci * MAX_WH
    ox1 = bx1 + off
    oy1 = by1 + off
    ox2 = bx2 + off
    oy2 = by2 + off
    avail = jnp.where(cs > SCORE_THRES, cs, NEG)
    area = (ox2 - ox1) * (oy2 - oy1)
    zero = jnp.zeros((5, BP), jnp.float32)
    f_ref[...] = jnp.concatenate(
        [avail, ox1, oy1, ox2, oy2, bx1, by1, bx2, by2, ci, area,
         zero], axis=0)


def _nms_kernel(f_ref, dets_ref, av_ref):
    av_ref[...] = f_ref[0]

    ridx = lax.broadcasted_iota(jnp.int32, (NROW, 128), 0)
    lidx = lax.broadcasted_iota(jnp.int32, (NROW, 128), 1)
    idxv = ridx * 128 + lidx
    lane = lax.broadcasted_iota(jnp.int32, (1, 128), 1)

    def body(it, _):
        av = av_ref[...]
        m = jnp.max(av)
        ok = m > 0.0
        wi = jnp.min(jnp.where(av == m, idxv, BIGI))
        sel = idxv == wi

        def pick(r):
            return jnp.max(jnp.where(sel, f_ref[r], NEG))

        rb1, rb2, rb3, rb4 = pick(5), pick(6), pick(7), pick(8)
        wcls = pick(9)
        woff = wcls * MAX_WH
        wx1 = rb1 + woff
        wy1 = rb2 + woff
        wx2 = rb3 + woff
        wy2 = rb4 + woff
        warea = (wx2 - wx1) * (wy2 - wy1)
        ix1 = jnp.maximum(wx1, f_ref[1])
        iy1 = jnp.maximum(wy1, f_ref[2])
        ix2 = jnp.minimum(wx2, f_ref[3])
        iy2 = jnp.minimum(wy2, f_ref[4])
        inter = jnp.maximum(ix2 - ix1, 0.0) * jnp.maximum(iy2 - iy1, 0.0)
        iou = inter / (warea + f_ref[10] - inter + 1e-9)
        supp = jnp.logical_or(jnp.logical_and(ok, iou > IOU_THRES), sel)
        av_ref[...] = jnp.where(supp, NEG, av)

        row = jnp.where(lane == 0, jnp.where(ok, rb1, 0.0), 0.0)
        row = jnp.where(lane == 1, jnp.where(ok, rb2, 0.0), row)
        row = jnp.where(lane == 2, jnp.where(ok, rb3, 0.0), row)
        row = jnp.where(lane == 3, jnp.where(ok, rb4, 0.0), row)
        row = jnp.where(lane == 4, jnp.where(ok, m, 0.0), row)
        row = jnp.where(lane == 5, jnp.where(ok, wcls, -1.0), row)
        dets_ref[pl.ds(it, 1), :] = row
        return 0

    lax.fori_loop(0, MAX_OBJ, body, 0)


def _prep(xp, interpret=False):
    return pl.pallas_call(
        _prep_kernel,
        grid=(NPAD // BP,),
        in_specs=[pl.BlockSpec((BP, 85), lambda i: (i, 0))],
        out_specs=pl.BlockSpec((16, BP), lambda i: (0, i)),
        out_shape=jax.ShapeDtypeStruct((16, NPAD), jnp.float32),
        interpret=interpret,
    )(xp)


def _run_nms(ft, interpret=False):
    return pl.pallas_call(
        _nms_kernel,
        out_shape=jax.ShapeDtypeStruct((MAX_OBJ, 128), jnp.float32),
        scratch_shapes=[pltpu.VMEM((NROW, 128), jnp.float32)],
        interpret=interpret,
    )(ft)


def kernel(x):
    xp = jnp.pad(x[0], ((0, NPAD - N_BOXES), (0, 0)))
    ft = _prep(xp).reshape(16, NROW, 128)[:12]
    dets = _run_nms(ft)
    return dets[None, :, :6]


# speedup vs baseline: 2.0101x; 1.2061x over previous
import jax
import jax.numpy as jnp
from jax import lax
from jax.experimental import pallas as pl
from jax.experimental.pallas import tpu as pltpu

MAX_OBJ = 100
IOU_THRES = 0.45
SCORE_THRES = 0.25
NC = 80
MAX_WH = 640.0
N_BOXES = 20000
NPAD = 20480
NROW = NPAD // 128
NEG = -1.0e30
BIGI = 1 << 30


BP = 2000
BPO = 2048


def _prep_kernel(x_ref, f_ref):
    xt = x_ref[...].T
    conf = xt[4:5, :]
    cls = xt[5:85, :]
    scores = conf * cls
    cs = jnp.max(scores, axis=0, keepdims=True)
    csub = lax.broadcasted_iota(jnp.int32, (NC, BP), 0).astype(
        jnp.float32)
    ci = (NC - 1) - jnp.max(
        jnp.where(scores == cs, (NC - 1) - csub, -1.0), axis=0,
        keepdims=True)
    bx1 = xt[0:1, :] - xt[2:3, :] * 0.5
    by1 = xt[1:2, :] - xt[3:4, :] * 0.5
    bx2 = xt[0:1, :] + xt[2:3, :] * 0.5
    by2 = xt[1:2, :] + xt[3:4, :] * 0.5
    off = ci * MAX_WH
    ox1 = bx1 + off
    oy1 = by1 + off
    ox2 = bx2 + off
    oy2 = by2 + off
    avail = jnp.where(cs > SCORE_THRES, cs, NEG)
    area = (ox2 - ox1) * (oy2 - oy1)
    zero = jnp.zeros((5, BP), jnp.float32)
    fb = jnp.concatenate(
        [avail, ox1, oy1, ox2, oy2, bx1, by1, bx2, by2, ci, area,
         zero], axis=0)
    f_ref[...] = jnp.concatenate(
        [fb, jnp.zeros((16, BPO - BP), jnp.float32)], axis=1)


def _nms_kernel(f_ref, dets_ref, av_ref):
    ridx = lax.broadcasted_iota(jnp.int32, (NROW, 128), 0)
    lidx = lax.broadcasted_iota(jnp.int32, (NROW, 128), 1)
    col = ridx * 128 + lidx
    blk = col // BPO
    off = col - blk * BPO
    valid = off < BP
    idxv = jnp.where(valid, blk * BP + off, BIGI)
    av_ref[...] = jnp.where(valid, f_ref[0], NEG)
    lane = lax.broadcasted_iota(jnp.int32, (1, 128), 1)

    def body(it, _):
        av = av_ref[...]
        m = jnp.max(av)
        ok = m > 0.0
        wi = jnp.min(jnp.where(av == m, idxv, BIGI))
        sel = idxv == wi

        def pick(r):
            return jnp.max(jnp.where(sel, f_ref[r], NEG))

        rb1, rb2, rb3, rb4 = pick(5), pick(6), pick(7), pick(8)
        wcls = pick(9)
        woff = wcls * MAX_WH
        wx1 = rb1 + woff
        wy1 = rb2 + woff
        wx2 = rb3 + woff
        wy2 = rb4 + woff
        warea = (wx2 - wx1) * (wy2 - wy1)
        ix1 = jnp.maximum(wx1, f_ref[1])
        iy1 = jnp.maximum(wy1, f_ref[2])
        ix2 = jnp.minimum(wx2, f_ref[3])
        iy2 = jnp.minimum(wy2, f_ref[4])
        inter = jnp.maximum(ix2 - ix1, 0.0) * jnp.maximum(iy2 - iy1, 0.0)
        iou = inter / (warea + f_ref[10] - inter + 1e-9)
        supp = jnp.logical_or(jnp.logical_and(ok, iou > IOU_THRES), sel)
        av_ref[...] = jnp.where(supp, NEG, av)

        row = jnp.where(lane == 0, jnp.where(ok, rb1, 0.0), 0.0)
        row = jnp.where(lane == 1, jnp.where(ok, rb2, 0.0), row)
        row = jnp.where(lane == 2, jnp.where(ok, rb3, 0.0), row)
        row = jnp.where(lane == 3, jnp.where(ok, rb4, 0.0), row)
        row = jnp.where(lane == 4, jnp.where(ok, m, 0.0), row)
        row = jnp.where(lane == 5, jnp.where(ok, wcls, -1.0), row)
        dets_ref[pl.ds(it, 1), :] = row
        return 0

    lax.fori_loop(0, MAX_OBJ, body, 0)


def _prep(xp, interpret=False):
    return pl.pallas_call(
        _prep_kernel,
        grid=(N_BOXES // BP,),
        in_specs=[pl.BlockSpec((BP, 85), lambda i: (i, 0))],
        out_specs=pl.BlockSpec((16, BPO), lambda i: (0, i)),
        out_shape=jax.ShapeDtypeStruct((16, NPAD), jnp.float32),
        interpret=interpret,
    )(xp)


def _run_nms(ft, interpret=False):
    return pl.pallas_call(
        _nms_kernel,
        out_shape=jax.ShapeDtypeStruct((MAX_OBJ, 128), jnp.float32),
        scratch_shapes=[pltpu.VMEM((NROW, 128), jnp.float32)],
        interpret=interpret,
    )(ft)


def kernel(x):
    ft = _prep(x[0]).reshape(16, NROW, 128)[:12]
    dets = _run_nms(ft)
    return dets[None, :, :6]
